# baseline (speedup 1.0000x reference)
import functools

import jax
import jax.numpy as jnp
from jax import lax
from jax.experimental import pallas as pl
from jax.experimental.pallas import tpu as pltpu

_STEEPNESS = 12.0
_NB = 4


def _fused_body(x1_ref, w_ref, x0_ref, out_ref):
    nb, R, C = x1_ref.shape
    D = x0_ref.shape[2]

    cr = jnp.sum(x1_ref[...], axis=1) * w_ref[...]

    lane = lax.broadcasted_iota(jnp.int32, (nb, C), 1)
    rank = jnp.zeros((nb, C), jnp.float32)
    for r in range(1, C):
        crj = pltpu.roll(cr, C - r, axis=1)
        gt = crj > cr
        tie = (crj == cr) & (lane >= C - r)
        rank = rank + (gt | tie).astype(jnp.float32)
    mask = jnp.exp((-_STEEPNESS / (C - 1)) * rank)

    out_ref[:, :, 0:D - 1] = x0_ref[:, :, 0:D - 1]
    out_ref[:, :, D - 1:D] = (
        x0_ref[:, :, D - 1:D] * mask[:, None, None, None, :]
    )


def kernel(x0, x1, boosting_weights):
    B, C, H, W, D = x0.shape
    xt = lax.transpose(x0, (0, 2, 4, 3, 1))
    x1t = lax.transpose(x1, (0, 2, 1))
    R = x1t.shape[1]
    w = boosting_weights.reshape(1, C)
    out = pl.pallas_call(
        _fused_body,
        grid=(B // _NB,),
        in_specs=[
            pl.BlockSpec((_NB, R, C), lambda i: (i, 0, 0)),
            pl.BlockSpec((1, C), lambda i: (0, 0)),
            pl.BlockSpec((_NB, H, D, W, C), lambda i: (i, 0, 0, 0, 0)),
        ],
        out_specs=pl.BlockSpec((_NB, H, D, W, C), lambda i: (i, 0, 0, 0, 0)),
        out_shape=jax.ShapeDtypeStruct((B, H, D, W, C), x0.dtype),
    )(x1t, w, xt)
    return lax.transpose(out, (0, 4, 1, 3, 2))
